# baseline (speedup 1.0000x reference)
import functools

import jax
import jax.numpy as jnp
from jax import lax
from jax.experimental import pallas as pl
from jax.experimental.pallas import tpu as pltpu
from jax.experimental.pallas import tpu_sc as plsc

N = 10000
E = 160000
IN = 256
H = 256
OUT = 128
NUM_RELS = 8
NUM_BASES = 4

NC = 2
NS = 16
BATCH = 128
NPAD = N + 16
ROWS_PER_TILE_Z = NPAD // NS
ROWS_PER_TILE_O = 624


SB = 8


def _make_sc_agg(nb, edge_split):
  assert nb % (2 * SB) == 0
  npairs = nb // (2 * SB)
  mesh = plsc.VectorSubcoreMesh(core_axis_name="c", subcore_axis_name="s")

  @functools.partial(
      pl.kernel,
      mesh=mesh,
      out_type=jax.ShapeDtypeStruct((NC, N, 128), jnp.float32),
      scratch_types=[
          pltpu.VMEM((SB, BATCH), jnp.int32),
          pltpu.VMEM((SB, BATCH), jnp.int32),
          pltpu.VMEM((SB, BATCH), jnp.int32),
          pltpu.VMEM((SB, BATCH), jnp.int32),
          pltpu.VMEM((SB, BATCH), jnp.int32),
          pltpu.VMEM((SB, BATCH), jnp.int32),
          pltpu.VMEM((BATCH,), jnp.int32),
          pltpu.VMEM((BATCH,), jnp.int32),
          pltpu.VMEM((BATCH, 128), jnp.float32),
          pltpu.VMEM((BATCH, 128), jnp.float32),
          pltpu.VMEM_SHARED((NPAD, 128), jnp.float32),
          pltpu.SemaphoreType.DMA,
          pltpu.SemaphoreType.DMA,
          pltpu.SemaphoreType.DMA,
          pltpu.SemaphoreType.DMA,
      ],
  )
  def k(src_hbm, dst_hbm, rel_hbm, hr_hbm, out_hbm,
        src_pa, rel_pa, dst_pa, src_pb, rel_pb, dst_pb,
        gix_a, gix_b, rows_a, rows_b,
        acc_sh, sem_a, sem_b, sem_ta, sem_tb):
    c = lax.axis_index("c")
    s = lax.axis_index("s")
    grp = c * NS + s if edge_split else s
    goff = jnp.int32(0) if edge_split else c * jnp.int32(NUM_RELS * N)

    def zrow(i, _):
      for j in range(128 // 16):
        rows_a[i, pl.ds(j * 16, 16)] = jnp.zeros((16,), jnp.float32)
      return _
    lax.fori_loop(0, BATCH, zrow, None)
    zbase = s * ROWS_PER_TILE_Z
    for kk in range(4):
      pltpu.sync_copy(rows_a, acc_sh.at[pl.ds(zbase + kk * BATCH, BATCH)])
    rem = ROWS_PER_TILE_Z - 4 * BATCH
    pltpu.sync_copy(rows_a.at[pl.ds(0, rem)],
                    acc_sh.at[pl.ds(zbase + 4 * BATCH, rem)])

    plsc.subcore_barrier()

    slots = ((gix_a, rows_a, sem_a), (gix_b, rows_b, sem_b))
    bufs = ((src_pa, rel_pa, dst_pa, sem_ta), (src_pb, rel_pb, dst_pb, sem_tb))

    def load_triples(sb, buf):
      sv, rv, dv, st = buf
      pltpu.async_copy(src_hbm.at[grp, pl.ds(sb * SB, SB)], sv, st)
      pltpu.async_copy(rel_hbm.at[grp, pl.ds(sb * SB, SB)], rv, st)
      pltpu.async_copy(dst_hbm.at[grp, pl.ds(sb * SB, SB)], dv, st)

    def wait_triples(sb, buf):
      sv, rv, dv, st = buf
      pltpu.make_async_copy(src_hbm.at[grp, pl.ds(sb * SB, SB)], sv, st).wait()
      pltpu.make_async_copy(rel_hbm.at[grp, pl.ds(sb * SB, SB)], rv, st).wait()
      pltpu.make_async_copy(dst_hbm.at[grp, pl.ds(sb * SB, SB)], dv, st).wait()

    def stage(buf, j, gix_v, rows_v, sem):
      sv, rv, _, _ = buf

      def gix(i, __):
        r = rv[j, pl.ds(i * 16, 16)]
        s_ = sv[j, pl.ds(i * 16, 16)]
        gix_v[pl.ds(i * 16, 16)] = (r & 7) * jnp.int32(N) + s_ + goff
        return __
      lax.fori_loop(0, BATCH // 16, gix, None, unroll=True)
      pltpu.async_copy(hr_hbm.at[gix_v], rows_v, sem)

    def drain(buf, j, gix_v, rows_v, sem):
      dv = buf[2]
      pltpu.make_async_copy(hr_hbm.at[gix_v], rows_v, sem).wait()
      pltpu.sync_copy(rows_v, acc_sh.at[dv.at[j]], add=True)

    load_triples(0, bufs[0])
    wait_triples(0, bufs[0])
    load_triples(1, bufs[1])
    stage(bufs[0], 0, *slots[0])

    def body(u, _):
      for h in range(2):
        cur = bufs[h]
        oth = bufs[1 - h]
        for j in range(SB):
          if j < SB - 1:
            stage(cur, j + 1, *slots[(j + 1) % 2])
          elif h == 0:
            wait_triples(2 * u + 1, oth)
            stage(oth, 0, *slots[0])
          else:
            @pl.when(u + 1 < npairs)
            def _xb():
              wait_triples(2 * u + 2, oth)
              stage(oth, 0, *slots[0])
          drain(cur, j, *slots[j % 2])

        @pl.when(u + 1 < npairs)
        def _pf():
          load_triples(2 * u + 2 + h, cur)
      return _
    lax.fori_loop(0, npairs, body, None)

    plsc.subcore_barrier()

    obase = s * ROWS_PER_TILE_O
    pltpu.sync_copy(acc_sh.at[pl.ds(obase, ROWS_PER_TILE_O)],
                    out_hbm.at[c, pl.ds(obase, ROWS_PER_TILE_O)])

    @pl.when(s == NS - 1)
    def _tail():
      tb = NS * ROWS_PER_TILE_O
      pltpu.sync_copy(acc_sh.at[pl.ds(tb, N - tb)],
                      out_hbm.at[c, pl.ds(tb, N - tb)])

  return k


def _round_up(a, b):
  return (a + b - 1) // b * b


NB12 = _round_up((E + NS * BATCH - 1) // (NS * BATCH), 2 * SB)
NB3 = _round_up((E + NC * NS * BATCH - 1) // (NC * NS * BATCH), 2 * SB)
_sc_agg_feat = _make_sc_agg(nb=NB12, edge_split=False)
_sc_agg_edge = _make_sc_agg(nb=NB3, edge_split=True)

BN = 400


def _mm_kernel(x_ref, v_ref, c_ref, r_ref, hr_ref, hs_ref, *, out_dim):
  halves = out_dim // 128
  x = x_ref[...]
  hb = jnp.dot(x, v_ref[...], preferred_element_type=jnp.float32)
  for c in range(halves):
    for r in range(NUM_RELS):
      acc = None
      for b in range(NUM_BASES):
        piece = hb[:, b * out_dim + c * 128: b * out_dim + c * 128 + 128]
        term = c_ref[r, b] * piece
        acc = term if acc is None else acc + term
      hr_ref[c * NUM_RELS + r] = acc
  for c in range(halves):
    hs_ref[c] = jnp.dot(x, r_ref[:, c * 128:(c + 1) * 128],
                        preferred_element_type=jnp.float32)


def _mm(h, vflat, cpad, r, out_dim):
  halves = out_dim // 128
  return pl.pallas_call(
      functools.partial(_mm_kernel, out_dim=out_dim),
      grid=(N // BN,),
      in_specs=[
          pl.BlockSpec((BN, IN), lambda i: (i, 0)),
          pl.BlockSpec((IN, NUM_BASES * out_dim), lambda i: (0, 0)),
          pl.BlockSpec((NUM_RELS, 128), lambda i: (0, 0)),
          pl.BlockSpec((IN, out_dim), lambda i: (0, 0)),
      ],
      out_specs=[
          pl.BlockSpec((halves * NUM_RELS, BN, 128), lambda i: (0, i, 0)),
          pl.BlockSpec((halves, BN, 128), lambda i: (0, i, 0)),
      ],
      out_shape=[
          jax.ShapeDtypeStruct((halves * NUM_RELS, N, 128), jnp.float32),
          jax.ShapeDtypeStruct((halves, N, 128), jnp.float32),
      ],
  )(h, vflat, cpad, r)


def _act_relu_kernel(agg_ref, hs_ref, out_ref):
  a = agg_ref[...]
  s = hs_ref[...]
  out_ref[...] = jnp.maximum(
      jnp.concatenate([a[0] + s[0], a[1] + s[1]], axis=-1), 0.0)


def _act_relu(agg, hs):
  return pl.pallas_call(
      _act_relu_kernel,
      grid=(N // BN,),
      in_specs=[
          pl.BlockSpec((2, BN, 128), lambda i: (0, i, 0)),
          pl.BlockSpec((2, BN, 128), lambda i: (0, i, 0)),
      ],
      out_specs=pl.BlockSpec((BN, 256), lambda i: (i, 0)),
      out_shape=jax.ShapeDtypeStruct((N, 256), jnp.float32),
  )(agg, hs)


def _act_softmax_kernel(agg_ref, hs_ref, out_ref):
  a = agg_ref[...]
  t = a[0] + a[1] + hs_ref[...]
  m = jnp.max(t, axis=-1, keepdims=True)
  e = jnp.exp(t - m)
  out_ref[...] = e / jnp.sum(e, axis=-1, keepdims=True)


def _act_softmax(agg, hs):
  return pl.pallas_call(
      _act_softmax_kernel,
      grid=(N // BN,),
      in_specs=[
          pl.BlockSpec((2, BN, 128), lambda i: (0, i, 0)),
          pl.BlockSpec((BN, 128), lambda i: (i, 0)),
      ],
      out_specs=pl.BlockSpec((BN, 128), lambda i: (i, 0)),
      out_shape=jax.ShapeDtypeStruct((N, OUT), jnp.float32),
  )(agg, hs)


def _pad_edges(a, epad, fill):
  return jnp.pad(a, (0, epad - E), constant_values=fill)


def kernel(x, adj_t, V1, C1, R1, V2, C2, R2, V3, C3, R3):
  src = adj_t[0]
  dst = adj_t[1]
  rel = adj_t[2]

  ep12 = NS * NB12 * BATCH
  src12 = _pad_edges(src, ep12, 0).reshape(NS, NB12, BATCH)
  dst12 = _pad_edges(dst, ep12, N).reshape(NS, NB12, BATCH)
  rel12 = _pad_edges(rel, ep12, 0).reshape(NS, NB12, BATCH)
  ep3 = NC * NS * NB3 * BATCH
  src3 = _pad_edges(src, ep3, 0).reshape(NC * NS, NB3, BATCH)
  dst3 = _pad_edges(dst, ep3, N).reshape(NC * NS, NB3, BATCH)
  rel3 = _pad_edges(rel, ep3, 0).reshape(NC * NS, NB3, BATCH)

  def prep(V, C):
    vflat = V.transpose(1, 0, 2).reshape(IN, NUM_BASES * V.shape[2])
    cpad = jnp.pad(C, ((0, 0), (0, 128 - NUM_BASES)))
    return vflat, cpad

  v1f, c1p = prep(V1, C1)
  v2f, c2p = prep(V2, C2)
  v3f, c3p = prep(V3, C3)

  h = x
  for vf, cp, r in ((v1f, c1p, R1), (v2f, c2p, R2)):
    hr, hs = _mm(h, vf, cp, r, 256)
    agg = _sc_agg_feat(src12, dst12, rel12, hr.reshape(2 * NUM_RELS * N, 128))
    h = _act_relu(agg, hs)

  hr3, hs3 = _mm(h, v3f, c3p, R3, 128)
  agg3 = _sc_agg_edge(src3, dst3, rel3, hr3.reshape(NUM_RELS * N, 128))
  return _act_softmax(agg3, hs3.reshape(N, 128))


# speedup vs baseline: 2.2049x; 2.2049x over previous
import functools

import jax
import jax.numpy as jnp
from jax import lax
from jax.experimental import pallas as pl
from jax.experimental.pallas import tpu as pltpu
from jax.experimental.pallas import tpu_sc as plsc

N = 10000
E = 160000
IN = 256
H = 256
OUT = 128
NUM_RELS = 8
NUM_BASES = 4

NC = 2
NS = 16
BATCH = 128
NPAD = N + 16
ROWS_PER_TILE_Z = NPAD // NS
ROWS_PER_TILE_O = 624


SB = 8


def _make_sc_agg(nb, edge_split):
  mesh = plsc.VectorSubcoreMesh(core_axis_name="c", subcore_axis_name="s")

  @functools.partial(
      pl.kernel,
      mesh=mesh,
      out_type=jax.ShapeDtypeStruct((NC, N, 128), jnp.float32),
      scratch_types=[
          pltpu.VMEM((BATCH,), jnp.int32),
          pltpu.VMEM((BATCH,), jnp.int32),
          pltpu.VMEM((BATCH,), jnp.int32),
          pltpu.VMEM((BATCH,), jnp.int32),
          pltpu.VMEM((BATCH, 128), jnp.float32),
          pltpu.VMEM((BATCH, 128), jnp.float32),
          pltpu.VMEM_SHARED((NPAD, 128), jnp.float32),
          pltpu.SemaphoreType.DMA,
          pltpu.SemaphoreType.DMA,
      ],
  )
  def k(gix_hbm, dst_hbm, hr_hbm, out_hbm,
        dst_a, dst_b, gix_a, gix_b, rows_a, rows_b,
        acc_sh, sem_a, sem_b):
    c = lax.axis_index("c")
    s = lax.axis_index("s")
    grp = c * NS + s if edge_split else s
    cc = jnp.int32(0) if edge_split else c

    def zrow(i, _):
      for j in range(128 // 16):
        rows_a[i, pl.ds(j * 16, 16)] = jnp.zeros((16,), jnp.float32)
      return _
    lax.fori_loop(0, BATCH, zrow, None)
    zbase = s * ROWS_PER_TILE_Z
    for kk in range(4):
      pltpu.sync_copy(rows_a, acc_sh.at[pl.ds(zbase + kk * BATCH, BATCH)])
    rem = ROWS_PER_TILE_Z - 4 * BATCH
    pltpu.sync_copy(rows_a.at[pl.ds(0, rem)],
                    acc_sh.at[pl.ds(zbase + 4 * BATCH, rem)])

    plsc.subcore_barrier()

    def stage(bb, gix_v, dst_v, rows_v, sem):
      pltpu.sync_copy(gix_hbm.at[cc, grp, bb], gix_v)
      pltpu.sync_copy(dst_hbm.at[grp, bb], dst_v)
      pltpu.async_copy(hr_hbm.at[gix_v], rows_v, sem)

    def drain(gix_v, dst_v, rows_v, sem):
      pltpu.make_async_copy(hr_hbm.at[gix_v], rows_v, sem).wait()
      pltpu.sync_copy(rows_v, acc_sh.at[dst_v], add=True)

    stage(0, gix_a, dst_a, rows_a, sem_a)

    def body(g, _):
      b0 = 2 * g

      @pl.when(b0 + 1 < nb)
      def _sb():
        stage(b0 + 1, gix_b, dst_b, rows_b, sem_b)
      drain(gix_a, dst_a, rows_a, sem_a)

      @pl.when(b0 + 2 < nb)
      def _sa():
        stage(b0 + 2, gix_a, dst_a, rows_a, sem_a)

      @pl.when(b0 + 1 < nb)
      def _db():
        drain(gix_b, dst_b, rows_b, sem_b)
      return _
    lax.fori_loop(0, (nb + 1) // 2, body, None)

    plsc.subcore_barrier()

    obase = s * ROWS_PER_TILE_O
    pltpu.sync_copy(acc_sh.at[pl.ds(obase, ROWS_PER_TILE_O)],
                    out_hbm.at[c, pl.ds(obase, ROWS_PER_TILE_O)])

    @pl.when(s == NS - 1)
    def _tail():
      tb = NS * ROWS_PER_TILE_O
      pltpu.sync_copy(acc_sh.at[pl.ds(tb, N - tb)],
                      out_hbm.at[c, pl.ds(tb, N - tb)])

  return k


def _round_up(a, b):
  return (a + b - 1) // b * b


NB12 = (E + NS * BATCH - 1) // (NS * BATCH)
NB3 = (E + NC * NS * BATCH - 1) // (NC * NS * BATCH)
_sc_agg_feat = _make_sc_agg(nb=NB12, edge_split=False)
_sc_agg_edge = _make_sc_agg(nb=NB3, edge_split=True)

BN = 400


def _gix_kernel(src_ref, rel_ref, out_ref, *, off):
  g = (rel_ref[...] & 7) * jnp.int32(N) + src_ref[...]
  out_ref[0] = g
  if out_ref.shape[0] > 1:
    out_ref[1] = g + jnp.int32(off)


def _gix(src_pad, rel_pad, ncore, off):
  g, nb = src_pad.shape[0], src_pad.shape[1]
  rows = g * nb
  out = pl.pallas_call(
      functools.partial(_gix_kernel, off=off),
      out_shape=jax.ShapeDtypeStruct((ncore, rows, BATCH), jnp.int32),
  )(src_pad.reshape(rows, BATCH), rel_pad.reshape(rows, BATCH))
  return out.reshape(ncore, g, nb, BATCH)


def _mm_kernel(x_ref, v_ref, c_ref, r_ref, hr_ref, hs_ref, *, out_dim):
  halves = out_dim // 128
  x = x_ref[...]
  hb = jnp.dot(x, v_ref[...], preferred_element_type=jnp.float32)
  for c in range(halves):
    for r in range(NUM_RELS):
      acc = None
      for b in range(NUM_BASES):
        piece = hb[:, b * out_dim + c * 128: b * out_dim + c * 128 + 128]
        term = c_ref[r, b] * piece
        acc = term if acc is None else acc + term
      hr_ref[c * NUM_RELS + r] = acc
  for c in range(halves):
    hs_ref[c] = jnp.dot(x, r_ref[:, c * 128:(c + 1) * 128],
                        preferred_element_type=jnp.float32)


def _mm(h, vflat, cpad, r, out_dim):
  halves = out_dim // 128
  return pl.pallas_call(
      functools.partial(_mm_kernel, out_dim=out_dim),
      grid=(N // BN,),
      in_specs=[
          pl.BlockSpec((BN, IN), lambda i: (i, 0)),
          pl.BlockSpec((IN, NUM_BASES * out_dim), lambda i: (0, 0)),
          pl.BlockSpec((NUM_RELS, 128), lambda i: (0, 0)),
          pl.BlockSpec((IN, out_dim), lambda i: (0, 0)),
      ],
      out_specs=[
          pl.BlockSpec((halves * NUM_RELS, BN, 128), lambda i: (0, i, 0)),
          pl.BlockSpec((halves, BN, 128), lambda i: (0, i, 0)),
      ],
      out_shape=[
          jax.ShapeDtypeStruct((halves * NUM_RELS, N, 128), jnp.float32),
          jax.ShapeDtypeStruct((halves, N, 128), jnp.float32),
      ],
  )(h, vflat, cpad, r)


def _act_relu_kernel(agg_ref, hs_ref, out_ref):
  a = agg_ref[...]
  s = hs_ref[...]
  out_ref[...] = jnp.maximum(
      jnp.concatenate([a[0] + s[0], a[1] + s[1]], axis=-1), 0.0)


def _act_relu(agg, hs):
  return pl.pallas_call(
      _act_relu_kernel,
      grid=(N // BN,),
      in_specs=[
          pl.BlockSpec((2, BN, 128), lambda i: (0, i, 0)),
          pl.BlockSpec((2, BN, 128), lambda i: (0, i, 0)),
      ],
      out_specs=pl.BlockSpec((BN, 256), lambda i: (i, 0)),
      out_shape=jax.ShapeDtypeStruct((N, 256), jnp.float32),
  )(agg, hs)


def _act_softmax_kernel(agg_ref, hs_ref, out_ref):
  a = agg_ref[...]
  t = a[0] + a[1] + hs_ref[...]
  m = jnp.max(t, axis=-1, keepdims=True)
  e = jnp.exp(t - m)
  out_ref[...] = e / jnp.sum(e, axis=-1, keepdims=True)


def _act_softmax(agg, hs):
  return pl.pallas_call(
      _act_softmax_kernel,
      grid=(N // BN,),
      in_specs=[
          pl.BlockSpec((2, BN, 128), lambda i: (0, i, 0)),
          pl.BlockSpec((BN, 128), lambda i: (i, 0)),
      ],
      out_specs=pl.BlockSpec((BN, 128), lambda i: (i, 0)),
      out_shape=jax.ShapeDtypeStruct((N, OUT), jnp.float32),
  )(agg, hs)


def _pad_edges(a, epad, fill):
  return jnp.pad(a, (0, epad - E), constant_values=fill)


def kernel(x, adj_t, V1, C1, R1, V2, C2, R2, V3, C3, R3):
  src = adj_t[0]
  dst = adj_t[1]
  rel = adj_t[2]

  ep12 = NS * NB12 * BATCH
  src12 = _pad_edges(src, ep12, 0).reshape(NS, NB12, BATCH)
  dst12 = _pad_edges(dst, ep12, N).reshape(NS, NB12, BATCH)
  rel12 = _pad_edges(rel, ep12, 0).reshape(NS, NB12, BATCH)
  ep3 = NC * NS * NB3 * BATCH
  src3 = _pad_edges(src, ep3, 0).reshape(NC * NS, NB3, BATCH)
  dst3 = _pad_edges(dst, ep3, N).reshape(NC * NS, NB3, BATCH)
  rel3 = _pad_edges(rel, ep3, 0).reshape(NC * NS, NB3, BATCH)

  def prep(V, C):
    vflat = V.transpose(1, 0, 2).reshape(IN, NUM_BASES * V.shape[2])
    cpad = jnp.pad(C, ((0, 0), (0, 128 - NUM_BASES)))
    return vflat, cpad

  v1f, c1p = prep(V1, C1)
  v2f, c2p = prep(V2, C2)
  v3f, c3p = prep(V3, C3)

  gix12 = _gix(src12, rel12, 2, NUM_RELS * N)
  gix3 = _gix(src3, rel3, 1, 0)

  h = x
  for vf, cp, r in ((v1f, c1p, R1), (v2f, c2p, R2)):
    hr, hs = _mm(h, vf, cp, r, 256)
    agg = _sc_agg_feat(gix12, dst12, hr.reshape(2 * NUM_RELS * N, 128))
    h = _act_relu(agg, hs)

  hr3, hs3 = _mm(h, v3f, c3p, R3, 128)
  agg3 = _sc_agg_edge(gix3, dst3, hr3.reshape(NUM_RELS * N, 128))
  return _act_softmax(agg3, hs3.reshape(N, 128))


# speedup vs baseline: 2.4438x; 1.1083x over previous
import functools

import jax
import jax.numpy as jnp
from jax import lax
from jax.experimental import pallas as pl
from jax.experimental.pallas import tpu as pltpu
from jax.experimental.pallas import tpu_sc as plsc

N = 10000
E = 160000
IN = 256
H = 256
OUT = 128
NUM_RELS = 8
NUM_BASES = 4

NC = 2
NS = 16
BATCH = 128
NPAD = N + 16
ROWS_PER_TILE_Z = NPAD // NS
ROWS_PER_TILE_O = 624


SB = 8


def _make_sc_agg(nb, edge_split):
  mesh = plsc.VectorSubcoreMesh(core_axis_name="c", subcore_axis_name="s")

  @functools.partial(
      pl.kernel,
      mesh=mesh,
      out_type=jax.ShapeDtypeStruct((NC, N, 128), jnp.float32),
      scratch_types=[
          pltpu.VMEM((BATCH,), jnp.int32),
          pltpu.VMEM((BATCH,), jnp.int32),
          pltpu.VMEM((BATCH,), jnp.int32),
          pltpu.VMEM((BATCH,), jnp.int32),
          pltpu.VMEM((BATCH,), jnp.int32),
          pltpu.VMEM((BATCH,), jnp.int32),
          pltpu.VMEM((BATCH, 128), jnp.float32),
          pltpu.VMEM((BATCH, 128), jnp.float32),
          pltpu.VMEM_SHARED((NPAD, 128), jnp.float32),
          pltpu.SemaphoreType.DMA,
          pltpu.SemaphoreType.DMA,
      ],
  )
  def k(src_hbm, dst_hbm, rel_hbm, hr_hbm, out_hbm,
        src_v, rel_v, dst_a, dst_b, gix_a, gix_b, rows_a, rows_b,
        acc_sh, sem_a, sem_b):
    c = lax.axis_index("c")
    s = lax.axis_index("s")
    grp = c * NS + s if edge_split else s
    goff = jnp.int32(0) if edge_split else c * jnp.int32(NUM_RELS * N)

    def zrow(i, _):
      for j in range(128 // 16):
        rows_a[i, pl.ds(j * 16, 16)] = jnp.zeros((16,), jnp.float32)
      return _
    lax.fori_loop(0, BATCH, zrow, None)
    zbase = s * ROWS_PER_TILE_Z
    for kk in range(4):
      pltpu.sync_copy(rows_a, acc_sh.at[pl.ds(zbase + kk * BATCH, BATCH)])
    rem = ROWS_PER_TILE_Z - 4 * BATCH
    pltpu.sync_copy(rows_a.at[pl.ds(0, rem)],
                    acc_sh.at[pl.ds(zbase + 4 * BATCH, rem)])

    plsc.subcore_barrier()

    def stage(bb, gix_v, dst_v, rows_v, sem):
      pltpu.sync_copy(src_hbm.at[grp, bb], src_v)
      pltpu.sync_copy(rel_hbm.at[grp, bb], rel_v)

      def gix(j, __):
        r = rel_v[pl.ds(j * 16, 16)]
        sv = src_v[pl.ds(j * 16, 16)]
        gix_v[pl.ds(j * 16, 16)] = (r & 7) * jnp.int32(N) + sv + goff
        return __
      lax.fori_loop(0, BATCH // 16, gix, None, unroll=True)
      pltpu.sync_copy(dst_hbm.at[grp, bb], dst_v)
      pltpu.async_copy(hr_hbm.at[gix_v], rows_v, sem)

    def drain(gix_v, dst_v, rows_v, sem):
      pltpu.make_async_copy(hr_hbm.at[gix_v], rows_v, sem).wait()
      pltpu.sync_copy(rows_v, acc_sh.at[dst_v], add=True)

    stage(0, gix_a, dst_a, rows_a, sem_a)

    def body(g, _):
      b0 = 2 * g

      @pl.when(b0 + 1 < nb)
      def _sb():
        stage(b0 + 1, gix_b, dst_b, rows_b, sem_b)
      drain(gix_a, dst_a, rows_a, sem_a)

      @pl.when(b0 + 2 < nb)
      def _sa():
        stage(b0 + 2, gix_a, dst_a, rows_a, sem_a)

      @pl.when(b0 + 1 < nb)
      def _db():
        drain(gix_b, dst_b, rows_b, sem_b)
      return _
    lax.fori_loop(0, (nb + 1) // 2, body, None)

    plsc.subcore_barrier()

    obase = s * ROWS_PER_TILE_O
    pltpu.sync_copy(acc_sh.at[pl.ds(obase, ROWS_PER_TILE_O)],
                    out_hbm.at[c, pl.ds(obase, ROWS_PER_TILE_O)])

    @pl.when(s == NS - 1)
    def _tail():
      tb = NS * ROWS_PER_TILE_O
      pltpu.sync_copy(acc_sh.at[pl.ds(tb, N - tb)],
                      out_hbm.at[c, pl.ds(tb, N - tb)])

  return k


def _round_up(a, b):
  return (a + b - 1) // b * b


NB12 = (E + NS * BATCH - 1) // (NS * BATCH)
NB3 = (E + NC * NS * BATCH - 1) // (NC * NS * BATCH)
_sc_agg_feat = _make_sc_agg(nb=NB12, edge_split=False)
_sc_agg_edge = _make_sc_agg(nb=NB3, edge_split=True)

BN = 400


def _mm_body(x, v_ref, c_ref, r_ref, hr_ref, hs_ref, out_dim):
  halves = out_dim // 128
  hb = jnp.dot(x, v_ref[...], preferred_element_type=jnp.float32)
  for c in range(halves):
    for r in range(NUM_RELS):
      acc = None
      for b in range(NUM_BASES):
        piece = hb[:, b * out_dim + c * 128: b * out_dim + c * 128 + 128]
        term = c_ref[r, b] * piece
        acc = term if acc is None else acc + term
      hr_ref[c * NUM_RELS + r] = acc
  for c in range(halves):
    hs_ref[c] = jnp.dot(x, r_ref[:, c * 128:(c + 1) * 128],
                        preferred_element_type=jnp.float32)


def _mm_kernel(x_ref, v_ref, c_ref, r_ref, hr_ref, hs_ref, *, out_dim):
  _mm_body(x_ref[...], v_ref, c_ref, r_ref, hr_ref, hs_ref, out_dim)


def _actmm_kernel(agg_ref, hsp_ref, v_ref, c_ref, r_ref, hr_ref, hs_ref, *,
                  out_dim):
  a = agg_ref[...]
  sp = hsp_ref[...]
  x = jnp.maximum(jnp.concatenate([a[0] + sp[0], a[1] + sp[1]], axis=-1), 0.0)
  _mm_body(x, v_ref, c_ref, r_ref, hr_ref, hs_ref, out_dim)


def _actmm(agg, hsp, vflat, cpad, r, out_dim):
  halves = out_dim // 128
  return pl.pallas_call(
      functools.partial(_actmm_kernel, out_dim=out_dim),
      grid=(N // BN,),
      in_specs=[
          pl.BlockSpec((2, BN, 128), lambda i: (0, i, 0)),
          pl.BlockSpec((2, BN, 128), lambda i: (0, i, 0)),
          pl.BlockSpec((IN, NUM_BASES * out_dim), lambda i: (0, 0)),
          pl.BlockSpec((NUM_RELS, 128), lambda i: (0, 0)),
          pl.BlockSpec((IN, out_dim), lambda i: (0, 0)),
      ],
      out_specs=[
          pl.BlockSpec((halves * NUM_RELS, BN, 128), lambda i: (0, i, 0)),
          pl.BlockSpec((halves, BN, 128), lambda i: (0, i, 0)),
      ],
      out_shape=[
          jax.ShapeDtypeStruct((halves * NUM_RELS, N, 128), jnp.float32),
          jax.ShapeDtypeStruct((halves, N, 128), jnp.float32),
      ],
  )(agg, hsp, vflat, cpad, r)


def _mm(h, vflat, cpad, r, out_dim):
  halves = out_dim // 128
  return pl.pallas_call(
      functools.partial(_mm_kernel, out_dim=out_dim),
      grid=(N // BN,),
      in_specs=[
          pl.BlockSpec((BN, IN), lambda i: (i, 0)),
          pl.BlockSpec((IN, NUM_BASES * out_dim), lambda i: (0, 0)),
          pl.BlockSpec((NUM_RELS, 128), lambda i: (0, 0)),
          pl.BlockSpec((IN, out_dim), lambda i: (0, 0)),
      ],
      out_specs=[
          pl.BlockSpec((halves * NUM_RELS, BN, 128), lambda i: (0, i, 0)),
          pl.BlockSpec((halves, BN, 128), lambda i: (0, i, 0)),
      ],
      out_shape=[
          jax.ShapeDtypeStruct((halves * NUM_RELS, N, 128), jnp.float32),
          jax.ShapeDtypeStruct((halves, N, 128), jnp.float32),
      ],
  )(h, vflat, cpad, r)


def _act_relu_kernel(agg_ref, hs_ref, out_ref):
  a = agg_ref[...]
  s = hs_ref[...]
  out_ref[...] = jnp.maximum(
      jnp.concatenate([a[0] + s[0], a[1] + s[1]], axis=-1), 0.0)


def _act_relu(agg, hs):
  return pl.pallas_call(
      _act_relu_kernel,
      grid=(N // BN,),
      in_specs=[
          pl.BlockSpec((2, BN, 128), lambda i: (0, i, 0)),
          pl.BlockSpec((2, BN, 128), lambda i: (0, i, 0)),
      ],
      out_specs=pl.BlockSpec((BN, 256), lambda i: (i, 0)),
      out_shape=jax.ShapeDtypeStruct((N, 256), jnp.float32),
  )(agg, hs)


def _act_softmax_kernel(agg_ref, hs_ref, out_ref):
  a = agg_ref[...]
  t = a[0] + a[1] + hs_ref[...]
  m = jnp.max(t, axis=-1, keepdims=True)
  e = jnp.exp(t - m)
  out_ref[...] = e / jnp.sum(e, axis=-1, keepdims=True)


def _act_softmax(agg, hs):
  return pl.pallas_call(
      _act_softmax_kernel,
      grid=(N // BN,),
      in_specs=[
          pl.BlockSpec((2, BN, 128), lambda i: (0, i, 0)),
          pl.BlockSpec((BN, 128), lambda i: (i, 0)),
      ],
      out_specs=pl.BlockSpec((BN, 128), lambda i: (i, 0)),
      out_shape=jax.ShapeDtypeStruct((N, OUT), jnp.float32),
  )(agg, hs)


def _pad_edges(a, epad, fill):
  return jnp.pad(a, (0, epad - E), constant_values=fill)


def kernel(x, adj_t, V1, C1, R1, V2, C2, R2, V3, C3, R3):
  src = adj_t[0]
  dst = adj_t[1]
  rel = adj_t[2]

  ep12 = NS * NB12 * BATCH
  src12 = _pad_edges(src, ep12, 0).reshape(NS, NB12, BATCH)
  dst12 = _pad_edges(dst, ep12, N).reshape(NS, NB12, BATCH)
  rel12 = _pad_edges(rel, ep12, 0).reshape(NS, NB12, BATCH)
  ep3 = NC * NS * NB3 * BATCH
  src3 = _pad_edges(src, ep3, 0).reshape(NC * NS, NB3, BATCH)
  dst3 = _pad_edges(dst, ep3, N).reshape(NC * NS, NB3, BATCH)
  rel3 = _pad_edges(rel, ep3, 0).reshape(NC * NS, NB3, BATCH)

  def prep(V, C):
    vflat = V.transpose(1, 0, 2).reshape(IN, NUM_BASES * V.shape[2])
    cpad = jnp.pad(C, ((0, 0), (0, 128 - NUM_BASES)))
    return vflat, cpad

  v1f, c1p = prep(V1, C1)
  v2f, c2p = prep(V2, C2)
  v3f, c3p = prep(V3, C3)

  hr1, hs1 = _mm(x, v1f, c1p, R1, 256)
  agg1 = _sc_agg_feat(src12, dst12, rel12, hr1.reshape(2 * NUM_RELS * N, 128))
  hr2, hs2 = _actmm(agg1, hs1, v2f, c2p, R2, 256)
  agg2 = _sc_agg_feat(src12, dst12, rel12, hr2.reshape(2 * NUM_RELS * N, 128))
  hr3, hs3 = _actmm(agg2, hs2, v3f, c3p, R3, 128)
  agg3 = _sc_agg_edge(src3, dst3, rel3, hr3.reshape(NUM_RELS * N, 128))
  return _act_softmax(agg3, hs3.reshape(N, 128))
